# baseline (speedup 1.0000x reference)
import jax
import jax.numpy as jnp
from jax import lax
from jax.experimental import pallas as pl
from jax.experimental.pallas import tpu as pltpu
from jax.experimental.pallas import tpu_sc as plsc

D = 1024
S = 2048
B = 4
NC, NS, L = 2, 16, 16
NW = NC * NS
POSW = S // NW
CR = 8
CPB = POSW // CR
NCHUNK = CPB * B
NBUF = 10
KAHEAD = 6
JPR = D // L


def _emb_body(ids_hbm, table_hbm, pos_hbm, out_hbm, idx_v, pos_v, rows_v,
              *sems):
    gsem = sems[:NBUF]
    osem = sems[NBUF:2 * NBUF]
    psem = sems[2 * NBUF]

    wid = lax.axis_index("s") * NC + lax.axis_index("c")

    pd = [None] * CPB
    pd[0] = pltpu.async_copy(pos_hbm.at[pl.ds(wid * POSW, CR)],
                             pos_v.at[0], psem)

    pltpu.sync_copy(ids_hbm.at[wid], idx_v)

    gd = [None] * NCHUNK
    sd = [None] * NCHUNK

    def idx_of(cc):
        return idx_v.at[cc]

    for cc in range(min(KAHEAD, NCHUNK)):
        gd[cc] = pltpu.async_copy(table_hbm.at[idx_of(cc)],
                                  rows_v.at[cc % NBUF], gsem[cc % NBUF])

    for cc in range(NCHUNK):
        c, b = divmod(cc, B)
        buf = cc % NBUF
        nxt = cc + KAHEAD
        if nxt < NCHUNK:
            if nxt >= NBUF:
                sd[nxt - NBUF].wait()
            gd[nxt] = pltpu.async_copy(table_hbm.at[idx_of(nxt)],
                                       rows_v.at[nxt % NBUF], gsem[nxt % NBUF])
        if b == 0:
            pd[c].wait()
            if c + 1 < CPB:
                pd[c + 1] = pltpu.async_copy(
                    pos_hbm.at[pl.ds(wid * POSW + (c + 1) * CR, CR)],
                    pos_v.at[(c + 1) % 2], psem)
        gd[cc].wait()

        def add_grp(i):
            r = lax.shift_right_logical(i, 6)
            off = (i & (JPR - 1)) * L
            plsc.addupdate(rows_v.at[buf, r, pl.ds(off, L)],
                           pos_v[c % 2, r, pl.ds(off, L)])

        plsc.parallel_loop(0, CR * JPR, unroll=8)(add_grp)

        base = wid * POSW + c * CR
        sd[cc] = pltpu.async_copy(rows_v.at[buf],
                                  out_hbm.at[b, pl.ds(base, CR)], osem[buf])

    for cc in range(max(0, NCHUNK - NBUF), NCHUNK):
        sd[cc].wait()


def kernel(input_ids, word_embeddings, position_embeddings):
    ids = (input_ids.astype(jnp.int32)
           .reshape(B, NW, CPB, CR)
           .transpose(1, 2, 0, 3)
           .reshape(NW, NCHUNK, CR))
    mesh = plsc.VectorSubcoreMesh(core_axis_name="c", subcore_axis_name="s",
                                  num_cores=NC, num_subcores=NS)
    out = pl.kernel(
        _emb_body,
        out_type=jax.ShapeDtypeStruct((B, S, D), jnp.float32),
        mesh=mesh,
        scratch_types=[
            pltpu.VMEM((NCHUNK, CR), jnp.int32),
            pltpu.VMEM((2, CR, D), jnp.float32),
            pltpu.VMEM((NBUF, CR, D), jnp.float32),
        ] + [pltpu.SemaphoreType.DMA] * (2 * NBUF + 1),
    )(ids, word_embeddings, position_embeddings)
    return out


# speedup vs baseline: 1.0218x; 1.0218x over previous
import jax
import jax.numpy as jnp
from jax import lax
from jax.experimental import pallas as pl
from jax.experimental.pallas import tpu as pltpu
from jax.experimental.pallas import tpu_sc as plsc

D = 1024
S = 2048
B = 4
NC, NS, L = 2, 16, 16
NW = NC * NS
POSW = S // NW
CR = 16
CPB = POSW // CR
NCHUNK = CPB * B
NBUF = 5
KAHEAD = 3
JPR = D // L


def _emb_body(ids_hbm, table_hbm, pos_hbm, out_hbm, idx_v, pos_v, rows_v,
              *sems):
    gsem = sems[:NBUF]
    osem = sems[NBUF:2 * NBUF]
    psem = sems[2 * NBUF]

    wid = lax.axis_index("s") * NC + lax.axis_index("c")

    pd = [None] * CPB
    pd[0] = pltpu.async_copy(pos_hbm.at[pl.ds(wid * POSW, CR)],
                             pos_v.at[0], psem)

    pltpu.sync_copy(ids_hbm.at[wid], idx_v)

    gd = [None] * NCHUNK
    sd = [None] * NCHUNK

    def idx_of(cc):
        return idx_v.at[cc]

    for cc in range(min(KAHEAD, NCHUNK)):
        gd[cc] = pltpu.async_copy(table_hbm.at[idx_of(cc)],
                                  rows_v.at[cc % NBUF], gsem[cc % NBUF])

    for cc in range(NCHUNK):
        c, b = divmod(cc, B)
        buf = cc % NBUF
        nxt = cc + KAHEAD
        if nxt < NCHUNK:
            if nxt >= NBUF:
                sd[nxt - NBUF].wait()
            gd[nxt] = pltpu.async_copy(table_hbm.at[idx_of(nxt)],
                                       rows_v.at[nxt % NBUF], gsem[nxt % NBUF])
        if b == 0:
            pd[c].wait()
            if c + 1 < CPB:
                pd[c + 1] = pltpu.async_copy(
                    pos_hbm.at[pl.ds(wid * POSW + (c + 1) * CR, CR)],
                    pos_v.at[(c + 1) % 2], psem)
        gd[cc].wait()

        def add_grp(i):
            r = lax.shift_right_logical(i, 6)
            off = (i & (JPR - 1)) * L
            plsc.addupdate(rows_v.at[buf, r, pl.ds(off, L)],
                           pos_v[c % 2, r, pl.ds(off, L)])

        plsc.parallel_loop(0, CR * JPR, unroll=8)(add_grp)

        base = wid * POSW + c * CR
        sd[cc] = pltpu.async_copy(rows_v.at[buf],
                                  out_hbm.at[b, pl.ds(base, CR)], osem[buf])

    for cc in range(max(0, NCHUNK - NBUF), NCHUNK):
        sd[cc].wait()


def kernel(input_ids, word_embeddings, position_embeddings):
    ids = (input_ids.astype(jnp.int32)
           .reshape(B, NW, CPB, CR)
           .transpose(1, 2, 0, 3)
           .reshape(NW, NCHUNK, CR))
    mesh = plsc.VectorSubcoreMesh(core_axis_name="c", subcore_axis_name="s",
                                  num_cores=NC, num_subcores=NS)
    out = pl.kernel(
        _emb_body,
        out_type=jax.ShapeDtypeStruct((B, S, D), jnp.float32),
        mesh=mesh,
        compiler_params=pltpu.CompilerParams(
            disable_bounds_checks=True, disable_semaphore_checks=True),
        scratch_types=[
            pltpu.VMEM((NCHUNK, CR), jnp.int32),
            pltpu.VMEM((2, CR, D), jnp.float32),
            pltpu.VMEM((NBUF, CR, D), jnp.float32),
        ] + [pltpu.SemaphoreType.DMA] * (2 * NBUF + 1),
    )(ids, word_embeddings, position_embeddings)
    return out
